# baseline (speedup 1.0000x reference)
import functools

import jax
import jax.numpy as jnp
from jax import lax
from jax.experimental import pallas as pl
from jax.experimental.pallas import tpu as pltpu
from jax.experimental.pallas import tpu_sc as plsc

E = 8
N = 8192
D = 1024

BLK = 128
NB = N // BLK + E
NPAD = NB * BLK

NC = 2
NS = 16
NW = NC * NS
TOK_W = N // NW
CH = 32
NBUF = 3
NCH = TOK_W // CH


def _permute_body(src_hbm, idx_hbm, out_hbm, *sc, gather):
    idx_b, row_b = sc[0:NBUF], sc[NBUF:2 * NBUF]
    isem, osem = sc[2 * NBUF:3 * NBUF], sc[3 * NBUF:4 * NBUF]
    wid = lax.axis_index("s") * NC + lax.axis_index("c")
    base = wid * TOK_W
    in_d = [None] * NCH
    out_d = [None] * NCH

    def issue_in(c):
        b = c % NBUF
        off = base + c * CH
        pltpu.sync_copy(idx_hbm.at[pl.ds(off, CH)], idx_b[b])
        if gather:
            in_d[c] = pltpu.async_copy(src_hbm.at[idx_b[b]], row_b[b], isem[b])
        else:
            in_d[c] = pltpu.async_copy(
                src_hbm.at[pl.ds(off, CH)], row_b[b], isem[b])

    for p in range(min(NBUF, NCH)):
        issue_in(p)
    for c in range(NCH):
        b = c % NBUF
        off = base + c * CH
        in_d[c].wait()
        if gather:
            out_d[c] = pltpu.async_copy(
                row_b[b], out_hbm.at[pl.ds(off, CH)], osem[b])
        else:
            out_d[c] = pltpu.async_copy(row_b[b], out_hbm.at[idx_b[b]], osem[b])
        n = c + NBUF
        if n < NCH:
            out_d[c].wait()
            issue_in(n)
    for c in range(max(NCH - NBUF, 0), NCH):
        out_d[c].wait()


@functools.cache
def _make_permute(out_rows, gather):
    mesh = plsc.VectorSubcoreMesh(
        core_axis_name="c", subcore_axis_name="s",
        num_cores=NC, num_subcores=NS)
    return pl.kernel(
        functools.partial(_permute_body, gather=gather),
        out_type=jax.ShapeDtypeStruct((out_rows, D), jnp.float32),
        mesh=mesh,
        scratch_types=(
            [pltpu.VMEM((CH,), jnp.int32) for _ in range(NBUF)]
            + [pltpu.VMEM((CH, D), jnp.float32) for _ in range(NBUF)]
            + [pltpu.SemaphoreType.DMA for _ in range(2 * NBUF)]
        ),
    )


def _scatter_rows(src, idx):
    return _make_permute(NPAD, False)(src, idx)


def _gather_rows(src, idx):
    return _make_permute(N, True)(src, idx)


MROWS = 64
MCOLS = 128


def _meta_body(s_ref, dst_ref, be_ref):
    s = s_ref[...]
    dst = jnp.zeros((MROWS, MCOLS), jnp.int32)
    tstart = lax.broadcasted_iota(jnp.int32, (1, MCOLS), 1) * BLK
    be = jnp.zeros((1, MCOLS), jnp.int32)
    bound = jnp.int32(0)
    up = (lax.broadcasted_iota(jnp.int32, (MCOLS, MCOLS), 0)
          <= lax.broadcasted_iota(jnp.int32, (MCOLS, MCOLS), 1)
          ).astype(jnp.float32)
    lo = (lax.broadcasted_iota(jnp.int32, (MROWS, MROWS), 1)
          < lax.broadcasted_iota(jnp.int32, (MROWS, MROWS), 0)
          ).astype(jnp.float32)
    for e in range(E):
        m = s == e
        mf = m.astype(jnp.float32)
        c = lax.dot_general(mf, up, (((1,), (0,)), ((), ())),
                            preferred_element_type=jnp.float32)
        rowtot = c[:, MCOLS - 1:MCOLS]
        rowoff = lax.dot_general(lo, rowtot, (((1,), (0,)), ((), ())),
                                 preferred_element_type=jnp.float32)
        rank = (c + rowoff).astype(jnp.int32) - 1
        cnt = jnp.sum(m.astype(jnp.int32))
        dst = dst + jnp.where(m, bound + rank, 0)
        bound = bound + ((cnt + BLK - 1) // BLK) * BLK
        be = be + (tstart >= bound).astype(jnp.int32)
    dst_ref[...] = dst
    be_ref[...] = jnp.minimum(be, E - 1)


_meta = pl.pallas_call(
    _meta_body,
    out_shape=(
        jax.ShapeDtypeStruct((MROWS, MCOLS), jnp.int32),
        jax.ShapeDtypeStruct((1, MCOLS), jnp.int32),
    ),
)


def _mm_body(be_ref, x_ref, w_ref, b_ref, o_ref):
    acc = lax.dot_general(
        x_ref[...], w_ref[0],
        dimension_numbers=(((1,), (1,)), ((), ())),
        preferred_element_type=jnp.float32)
    o_ref[...] = acc + b_ref[0]


_grouped_mm = pl.pallas_call(
    _mm_body,
    grid_spec=pltpu.PrefetchScalarGridSpec(
        num_scalar_prefetch=1,
        grid=(NB,),
        in_specs=[
            pl.BlockSpec((BLK, D), lambda i, be: (i, 0)),
            pl.BlockSpec((1, D, D), lambda i, be: (be[i], 0, 0)),
            pl.BlockSpec((1, 1, D), lambda i, be: (be[i], 0, 0)),
        ],
        out_specs=pl.BlockSpec((BLK, D), lambda i, be: (i, 0)),
    ),
    out_shape=jax.ShapeDtypeStruct((NPAD, D), jnp.float32),
)


def kernel(x, split, W, b):
    split = split.astype(jnp.int32)
    dst2, be2 = _meta(split.reshape(MROWS, MCOLS))
    dst = dst2.reshape(N)
    block_expert = be2.reshape(MCOLS)

    xs = _scatter_rows(x, dst)
    ys = _grouped_mm(block_expert, xs, W, b[:, None, :])
    y = _gather_rows(ys, dst)
    return y


# speedup vs baseline: 1.2208x; 1.2208x over previous
import functools

import jax
import jax.numpy as jnp
from jax import lax
from jax.experimental import pallas as pl
from jax.experimental.pallas import tpu as pltpu
from jax.experimental.pallas import tpu_sc as plsc

E = 8
N = 8192
D = 1024

BLK = 256
NB = N // BLK + E
NPAD = NB * BLK

NC = 2
NS = 16
NW = NC * NS
TOK_W = N // NW
CH = 32
NBUF = 3
NCH = TOK_W // CH


def _permute_body(src_hbm, idx_hbm, out_hbm, *sc, gather):
    idx_b, row_b = sc[0:NBUF], sc[NBUF:2 * NBUF]
    isem, osem = sc[2 * NBUF:3 * NBUF], sc[3 * NBUF:4 * NBUF]
    wid = lax.axis_index("s") * NC + lax.axis_index("c")
    base = wid * TOK_W
    in_d = [None] * NCH
    out_d = [None] * NCH

    def issue_in(c):
        b = c % NBUF
        off = base + c * CH
        pltpu.sync_copy(idx_hbm.at[pl.ds(off, CH)], idx_b[b])
        if gather:
            in_d[c] = pltpu.async_copy(src_hbm.at[idx_b[b]], row_b[b], isem[b])
        else:
            in_d[c] = pltpu.async_copy(
                src_hbm.at[pl.ds(off, CH)], row_b[b], isem[b])

    for p in range(min(NBUF, NCH)):
        issue_in(p)
    for c in range(NCH):
        b = c % NBUF
        off = base + c * CH
        in_d[c].wait()
        if gather:
            out_d[c] = pltpu.async_copy(
                row_b[b], out_hbm.at[pl.ds(off, CH)], osem[b])
        else:
            out_d[c] = pltpu.async_copy(row_b[b], out_hbm.at[idx_b[b]], osem[b])
        n = c + NBUF
        if n < NCH:
            out_d[c].wait()
            issue_in(n)
    for c in range(max(NCH - NBUF, 0), NCH):
        out_d[c].wait()


@functools.cache
def _make_permute(out_rows, gather):
    mesh = plsc.VectorSubcoreMesh(
        core_axis_name="c", subcore_axis_name="s",
        num_cores=NC, num_subcores=NS)
    return pl.kernel(
        functools.partial(_permute_body, gather=gather),
        out_type=jax.ShapeDtypeStruct((out_rows, D), jnp.float32),
        mesh=mesh,
        scratch_types=(
            [pltpu.VMEM((CH,), jnp.int32) for _ in range(NBUF)]
            + [pltpu.VMEM((CH, D), jnp.float32) for _ in range(NBUF)]
            + [pltpu.SemaphoreType.DMA for _ in range(2 * NBUF)]
        ),
    )


def _scatter_rows(src, idx):
    return _make_permute(NPAD, False)(src, idx)


def _gather_rows(src, idx):
    return _make_permute(N, True)(src, idx)


MROWS = 64
MCOLS = 128


def _meta_body(s_ref, dst_ref, be_ref):
    s = s_ref[...]
    dst = jnp.zeros((MROWS, MCOLS), jnp.int32)
    tstart = lax.broadcasted_iota(jnp.int32, (1, MCOLS), 1) * BLK
    be = jnp.zeros((1, MCOLS), jnp.int32)
    bound = jnp.int32(0)
    up = (lax.broadcasted_iota(jnp.int32, (MCOLS, MCOLS), 0)
          <= lax.broadcasted_iota(jnp.int32, (MCOLS, MCOLS), 1)
          ).astype(jnp.float32)
    lo = (lax.broadcasted_iota(jnp.int32, (MROWS, MROWS), 1)
          < lax.broadcasted_iota(jnp.int32, (MROWS, MROWS), 0)
          ).astype(jnp.float32)
    for e in range(E):
        m = s == e
        mf = m.astype(jnp.float32)
        c = lax.dot_general(mf, up, (((1,), (0,)), ((), ())),
                            preferred_element_type=jnp.float32)
        rowtot = c[:, MCOLS - 1:MCOLS]
        rowoff = lax.dot_general(lo, rowtot, (((1,), (0,)), ((), ())),
                                 preferred_element_type=jnp.float32)
        rank = (c + rowoff).astype(jnp.int32) - 1
        cnt = jnp.sum(m.astype(jnp.int32))
        dst = dst + jnp.where(m, bound + rank, 0)
        bound = bound + ((cnt + BLK - 1) // BLK) * BLK
        be = be + (tstart >= bound).astype(jnp.int32)
    dst_ref[...] = dst
    be_ref[...] = jnp.minimum(be, E - 1)


_meta = pl.pallas_call(
    _meta_body,
    out_shape=(
        jax.ShapeDtypeStruct((MROWS, MCOLS), jnp.int32),
        jax.ShapeDtypeStruct((1, MCOLS), jnp.int32),
    ),
)


def _mm_body(be_ref, x_ref, w_ref, b_ref, o_ref):
    acc = lax.dot_general(
        x_ref[...], w_ref[0],
        dimension_numbers=(((1,), (1,)), ((), ())),
        preferred_element_type=jnp.float32)
    o_ref[...] = acc + b_ref[0]


_grouped_mm = pl.pallas_call(
    _mm_body,
    grid_spec=pltpu.PrefetchScalarGridSpec(
        num_scalar_prefetch=1,
        grid=(NB,),
        in_specs=[
            pl.BlockSpec((BLK, D), lambda i, be: (i, 0)),
            pl.BlockSpec((1, D, D), lambda i, be: (be[i], 0, 0)),
            pl.BlockSpec((1, 1, D), lambda i, be: (be[i], 0, 0)),
        ],
        out_specs=pl.BlockSpec((BLK, D), lambda i, be: (i, 0)),
    ),
    out_shape=jax.ShapeDtypeStruct((NPAD, D), jnp.float32),
)


def kernel(x, split, W, b):
    split = split.astype(jnp.int32)
    dst2, be2 = _meta(split.reshape(MROWS, MCOLS))
    dst = dst2.reshape(N)
    block_expert = be2.reshape(MCOLS)

    xs = _scatter_rows(x, dst)
    ys = _grouped_mm(block_expert, xs, W, b[:, None, :])
    y = _gather_rows(ys, dst)
    return y


# speedup vs baseline: 1.2780x; 1.0468x over previous
import functools

import jax
import jax.numpy as jnp
from jax import lax
from jax.experimental import pallas as pl
from jax.experimental.pallas import tpu as pltpu
from jax.experimental.pallas import tpu_sc as plsc

E = 8
N = 8192
D = 1024

BLK = 512
NB = N // BLK + E
NPAD = NB * BLK

NC = 2
NS = 16
NW = NC * NS
TOK_W = N // NW
CH = 32
NBUF = 3
NCH = TOK_W // CH


def _permute_body(src_hbm, idx_hbm, out_hbm, *sc, gather):
    idx_b, row_b = sc[0:NBUF], sc[NBUF:2 * NBUF]
    isem, osem = sc[2 * NBUF:3 * NBUF], sc[3 * NBUF:4 * NBUF]
    wid = lax.axis_index("s") * NC + lax.axis_index("c")
    base = wid * TOK_W
    in_d = [None] * NCH
    out_d = [None] * NCH

    def issue_in(c):
        b = c % NBUF
        off = base + c * CH
        pltpu.sync_copy(idx_hbm.at[pl.ds(off, CH)], idx_b[b])
        if gather:
            in_d[c] = pltpu.async_copy(src_hbm.at[idx_b[b]], row_b[b], isem[b])
        else:
            in_d[c] = pltpu.async_copy(
                src_hbm.at[pl.ds(off, CH)], row_b[b], isem[b])

    for p in range(min(NBUF, NCH)):
        issue_in(p)
    for c in range(NCH):
        b = c % NBUF
        off = base + c * CH
        in_d[c].wait()
        if gather:
            out_d[c] = pltpu.async_copy(
                row_b[b], out_hbm.at[pl.ds(off, CH)], osem[b])
        else:
            out_d[c] = pltpu.async_copy(row_b[b], out_hbm.at[idx_b[b]], osem[b])
        n = c + NBUF
        if n < NCH:
            out_d[c].wait()
            issue_in(n)
    for c in range(max(NCH - NBUF, 0), NCH):
        out_d[c].wait()


@functools.cache
def _make_permute(out_rows, gather):
    mesh = plsc.VectorSubcoreMesh(
        core_axis_name="c", subcore_axis_name="s",
        num_cores=NC, num_subcores=NS)
    return pl.kernel(
        functools.partial(_permute_body, gather=gather),
        out_type=jax.ShapeDtypeStruct((out_rows, D), jnp.float32),
        mesh=mesh,
        scratch_types=(
            [pltpu.VMEM((CH,), jnp.int32) for _ in range(NBUF)]
            + [pltpu.VMEM((CH, D), jnp.float32) for _ in range(NBUF)]
            + [pltpu.SemaphoreType.DMA for _ in range(2 * NBUF)]
        ),
    )


def _scatter_rows(src, idx):
    return _make_permute(NPAD, False)(src, idx)


def _gather_rows(src, idx):
    return _make_permute(N, True)(src, idx)


MROWS = 64
MCOLS = 128


def _meta_body(s_ref, dst_ref, be_ref):
    s = s_ref[...]
    dst = jnp.zeros((MROWS, MCOLS), jnp.int32)
    tstart = lax.broadcasted_iota(jnp.int32, (1, MCOLS), 1) * BLK
    be = jnp.zeros((1, MCOLS), jnp.int32)
    bound = jnp.int32(0)
    up = (lax.broadcasted_iota(jnp.int32, (MCOLS, MCOLS), 0)
          <= lax.broadcasted_iota(jnp.int32, (MCOLS, MCOLS), 1)
          ).astype(jnp.float32)
    lo = (lax.broadcasted_iota(jnp.int32, (MROWS, MROWS), 1)
          < lax.broadcasted_iota(jnp.int32, (MROWS, MROWS), 0)
          ).astype(jnp.float32)
    for e in range(E):
        m = s == e
        mf = m.astype(jnp.float32)
        c = lax.dot_general(mf, up, (((1,), (0,)), ((), ())),
                            preferred_element_type=jnp.float32)
        rowtot = c[:, MCOLS - 1:MCOLS]
        rowoff = lax.dot_general(lo, rowtot, (((1,), (0,)), ((), ())),
                                 preferred_element_type=jnp.float32)
        rank = (c + rowoff).astype(jnp.int32) - 1
        cnt = jnp.sum(m.astype(jnp.int32))
        dst = dst + jnp.where(m, bound + rank, 0)
        bound = bound + ((cnt + BLK - 1) // BLK) * BLK
        be = be + (tstart >= bound).astype(jnp.int32)
    dst_ref[...] = dst
    be_ref[...] = jnp.minimum(be, E - 1)


_meta = pl.pallas_call(
    _meta_body,
    out_shape=(
        jax.ShapeDtypeStruct((MROWS, MCOLS), jnp.int32),
        jax.ShapeDtypeStruct((1, MCOLS), jnp.int32),
    ),
)


def _mm_body(be_ref, x_ref, w_ref, b_ref, o_ref):
    acc = lax.dot_general(
        x_ref[...], w_ref[0],
        dimension_numbers=(((1,), (1,)), ((), ())),
        preferred_element_type=jnp.float32)
    o_ref[...] = (acc + b_ref[0]).astype(o_ref.dtype)


_grouped_mm = pl.pallas_call(
    _mm_body,
    grid_spec=pltpu.PrefetchScalarGridSpec(
        num_scalar_prefetch=1,
        grid=(NB,),
        in_specs=[
            pl.BlockSpec((BLK, D), lambda i, be: (i, 0)),
            pl.BlockSpec((1, D, D), lambda i, be: (be[i], 0, 0)),
            pl.BlockSpec((1, 1, D), lambda i, be: (be[i], 0, 0)),
        ],
        out_specs=pl.BlockSpec((BLK, D), lambda i, be: (i, 0)),
    ),
    out_shape=jax.ShapeDtypeStruct((NPAD, D), jnp.float32),
)


def kernel(x, split, W, b):
    split = split.astype(jnp.int32)
    dst2, be2 = _meta(split.reshape(MROWS, MCOLS))
    dst = dst2.reshape(N)
    block_expert = be2.reshape(MCOLS)

    xs = _scatter_rows(x, dst)
    ys = _grouped_mm(block_expert, xs, W, b[:, None, :])
    y = _gather_rows(ys, dst)
    return y


# speedup vs baseline: 1.2966x; 1.0146x over previous
import functools

import jax
import jax.numpy as jnp
from jax import lax
from jax.experimental import pallas as pl
from jax.experimental.pallas import tpu as pltpu
from jax.experimental.pallas import tpu_sc as plsc

E = 8
N = 8192
D = 1024

BLK = 512
NB = (N - E) // BLK + E
NPAD = NB * BLK

NC = 2
NS = 16
NW = NC * NS
TOK_W = N // NW
CH = 32
NBUF = 3
NCH = TOK_W // CH


def _permute_body(src_hbm, idx_hbm, out_hbm, *sc, gather):
    idx_b, row_b = sc[0:NBUF], sc[NBUF:2 * NBUF]
    isem, osem = sc[2 * NBUF:3 * NBUF], sc[3 * NBUF:4 * NBUF]
    wid = lax.axis_index("s") * NC + lax.axis_index("c")
    base = wid * TOK_W
    in_d = [None] * NCH
    out_d = [None] * NCH

    def issue_in(c):
        b = c % NBUF
        off = base + c * CH
        pltpu.sync_copy(idx_hbm.at[pl.ds(off, CH)], idx_b[b])
        if gather:
            in_d[c] = pltpu.async_copy(src_hbm.at[idx_b[b]], row_b[b], isem[b])
        else:
            in_d[c] = pltpu.async_copy(
                src_hbm.at[pl.ds(off, CH)], row_b[b], isem[b])

    for p in range(min(NBUF, NCH)):
        issue_in(p)
    for c in range(NCH):
        b = c % NBUF
        off = base + c * CH
        in_d[c].wait()
        if gather:
            out_d[c] = pltpu.async_copy(
                row_b[b], out_hbm.at[pl.ds(off, CH)], osem[b])
        else:
            out_d[c] = pltpu.async_copy(row_b[b], out_hbm.at[idx_b[b]], osem[b])
        n = c + NBUF
        if n < NCH:
            out_d[c].wait()
            issue_in(n)
    for c in range(max(NCH - NBUF, 0), NCH):
        out_d[c].wait()


@functools.cache
def _make_permute(out_rows, gather):
    mesh = plsc.VectorSubcoreMesh(
        core_axis_name="c", subcore_axis_name="s",
        num_cores=NC, num_subcores=NS)
    return pl.kernel(
        functools.partial(_permute_body, gather=gather),
        out_type=jax.ShapeDtypeStruct((out_rows, D), jnp.float32),
        mesh=mesh,
        scratch_types=(
            [pltpu.VMEM((CH,), jnp.int32) for _ in range(NBUF)]
            + [pltpu.VMEM((CH, D), jnp.float32) for _ in range(NBUF)]
            + [pltpu.SemaphoreType.DMA for _ in range(2 * NBUF)]
        ),
    )


def _scatter_rows(src, idx):
    return _make_permute(NPAD, False)(src, idx)


def _gather_rows(src, idx):
    return _make_permute(N, True)(src, idx)


MROWS = 64
MCOLS = 128


def _meta_body(s_ref, dst_ref, be_ref):
    s = s_ref[...]
    dst = jnp.zeros((MROWS, MCOLS), jnp.int32)
    tstart = lax.broadcasted_iota(jnp.int32, (1, MCOLS), 1) * BLK
    be = jnp.zeros((1, MCOLS), jnp.int32)
    bound = jnp.int32(0)
    up = (lax.broadcasted_iota(jnp.int32, (MCOLS, MCOLS), 0)
          <= lax.broadcasted_iota(jnp.int32, (MCOLS, MCOLS), 1)
          ).astype(jnp.float32)
    lo = (lax.broadcasted_iota(jnp.int32, (MROWS, MROWS), 1)
          < lax.broadcasted_iota(jnp.int32, (MROWS, MROWS), 0)
          ).astype(jnp.float32)
    for e in range(E):
        m = s == e
        mf = m.astype(jnp.float32)
        c = lax.dot_general(mf, up, (((1,), (0,)), ((), ())),
                            preferred_element_type=jnp.float32)
        rowtot = c[:, MCOLS - 1:MCOLS]
        rowoff = lax.dot_general(lo, rowtot, (((1,), (0,)), ((), ())),
                                 preferred_element_type=jnp.float32)
        rank = (c + rowoff).astype(jnp.int32) - 1
        cnt = jnp.sum(m.astype(jnp.int32))
        dst = dst + jnp.where(m, bound + rank, 0)
        bound = bound + ((cnt + BLK - 1) // BLK) * BLK
        be = be + (tstart >= bound).astype(jnp.int32)
    dst_ref[...] = dst
    be_ref[...] = jnp.minimum(be, E - 1)


_meta = pl.pallas_call(
    _meta_body,
    out_shape=(
        jax.ShapeDtypeStruct((MROWS, MCOLS), jnp.int32),
        jax.ShapeDtypeStruct((1, MCOLS), jnp.int32),
    ),
)


def _mm_body(be_ref, x_ref, w_ref, b_ref, o_ref):
    acc = lax.dot_general(
        x_ref[...], w_ref[0],
        dimension_numbers=(((1,), (1,)), ((), ())),
        preferred_element_type=jnp.float32)
    o_ref[...] = (acc + b_ref[0]).astype(o_ref.dtype)


_grouped_mm = pl.pallas_call(
    _mm_body,
    grid_spec=pltpu.PrefetchScalarGridSpec(
        num_scalar_prefetch=1,
        grid=(NB,),
        in_specs=[
            pl.BlockSpec((BLK, D), lambda i, be: (i, 0)),
            pl.BlockSpec((1, D, D), lambda i, be: (be[i], 0, 0)),
            pl.BlockSpec((1, 1, D), lambda i, be: (be[i], 0, 0)),
        ],
        out_specs=pl.BlockSpec((BLK, D), lambda i, be: (i, 0)),
    ),
    out_shape=jax.ShapeDtypeStruct((NPAD, D), jnp.float32),
)


def kernel(x, split, W, b):
    split = split.astype(jnp.int32)
    dst2, be2 = _meta(split.reshape(MROWS, MCOLS))
    dst = dst2.reshape(N)
    block_expert = be2.reshape(MCOLS)

    xs = _scatter_rows(x, dst)
    ys = _grouped_mm(block_expert, xs, W, b[:, None, :])
    y = _gather_rows(ys, dst)
    return y
